# baseline (speedup 1.0000x reference)
import functools

import jax
import jax.numpy as jnp
from jax import lax
from jax.experimental import pallas as pl
from jax.experimental.pallas import tpu as pltpu
from jax.experimental.pallas import tpu_sc as plsc

_NC = 2
_NS = 16
_L = 16


def _splat_i32(val):
    return jnp.full((_L,), val, dtype=jnp.int32)


_EXP2_P = (1.0000000491770782, 0.6931470026480493, 0.24022224204392606,
           0.05550712234706259, 0.009670180643285915, 0.0013260914912831122)
_MAGIC = 12582912.0
_LOG2E = 1.4426950408889634


def _sigmoid(z):
    u = z * (-_LOG2E)
    u = jnp.minimum(jnp.maximum(u, -126.0), 126.0)
    m = u + _MAGIC
    k = m - _MAGIC
    ki = lax.bitcast_convert_type(m, jnp.int32) - jnp.int32(0x4B400000)
    f = u - k
    r = jnp.float32(_EXP2_P[5])
    for i in range(4, -1, -1):
        r = r * f + jnp.float32(_EXP2_P[i])
    e = lax.bitcast_convert_type(
        lax.bitcast_convert_type(r, jnp.int32) + lax.shift_left(ki, 23),
        jnp.float32)
    d = 1.0 + e
    rc = lax.bitcast_convert_type(
        jnp.int32(0x7EF311C3) - lax.bitcast_convert_type(d, jnp.int32),
        jnp.float32)
    for _ in range(3):
        rc = rc * (2.0 - d * rc)
    return rc


def _sc_body(nrows, ngroups, x_hbm, consts_hbm, out_hbm, xv, cv, lut, ov, sem):
    wid = lax.axis_index("s") * _NC + lax.axis_index("c")
    xcp = pltpu.async_copy(
        x_hbm.at[pl.ds(wid * (nrows * 15), nrows * 15)], xv, sem)
    pltpu.sync_copy(consts_hbm, cv)

    def splat_c(r, c):
        return plsc.load_gather(cv, [_splat_i32(r * 32 + c)])

    w6 = [splat_c(18, j) for j in range(6)]
    vsp = []
    for d in range(12):
        acc = w6[0] * splat_c(12, d)
        for j in range(1, 6):
            acc = acc + w6[j] * splat_c(12 + j, d)
        vsp.append(acc)

    for half in range(2):
        ss = jnp.zeros((_L,), jnp.float32)
        aa = jnp.zeros((_L,), jnp.float32)
        for d in range(6):
            ss = ss + vsp[d] * cv[pl.ds(d * 32 + half * _L, _L)]
            aa = aa + vsp[6 + d] * cv[pl.ds((6 + d) * 32 + half * _L, _L)]
        lut[pl.ds(half * _L, _L)] = ss
        lut[pl.ds(32 + half * _L, _L)] = aa

    plsc.subcore_barrier()

    c16 = splat_c(18, 25)
    for i in range(6):
        c16 = c16 + splat_c(18, i) * splat_c(18, 19 + i)
    wd = [splat_c(18, 6 + k) for k in range(13)]

    lane15 = jax.lax.iota(jnp.int32, _L) * 15
    xcp.wait()

    _UNROLL = 4

    def group(gq, carry):
        for u in range(_UNROLL):
            g = gq * _UNROLL + u
            fid = lane15 + g * (_L * 15)
            si = plsc.load_gather(xv, [fid + 13]).astype(jnp.int32)
            ai = plsc.load_gather(xv, [fid + 14]).astype(jnp.int32)
            t = [plsc.load_gather(xv, [fid + k]) * wd[k] for k in range(13)]
            t.append(plsc.load_gather(lut, [si]))
            t.append(plsc.load_gather(lut, [ai + 32]))
            t.append(c16)
            while len(t) > 1:
                t = [t[i] + t[i + 1] for i in range(0, len(t) - 1, 2)] + (
                    [t[-1]] if len(t) % 2 else [])
            z = t[0]
            ov[pl.ds(g * _L, _L)] = _sigmoid(z)
        return carry

    lax.fori_loop(0, ngroups // _UNROLL, group, 0)
    pltpu.sync_copy(ov, out_hbm.at[pl.ds(wid * nrows, nrows)])


def kernel(x, site_table, app_table, fusion_W, fusion_b, log_W, log_b):
    B = x.shape[0]
    nw = _NC * _NS
    nrows = B // nw
    ngroups = nrows // _L
    assert nrows * nw == B and ngroups * _L == nrows and ngroups % 4 == 0

    stT = jnp.zeros((6, 32), jnp.float32).at[:, :24].set(site_table.T)
    atT = app_table.T.astype(jnp.float32)
    fWp = jnp.pad(fusion_W.astype(jnp.float32), ((0, 0), (0, 20)))
    wrow = jnp.concatenate(
        [log_W[0].astype(jnp.float32), fusion_b.astype(jnp.float32),
         log_b.astype(jnp.float32), jnp.zeros((6,), jnp.float32)])
    consts = jnp.concatenate([stT, atT, fWp, wrow[None, :]], axis=0).reshape(-1)

    xflat = x.astype(jnp.float32).reshape(-1)

    run = pl.kernel(
        functools.partial(_sc_body, nrows, ngroups),
        out_type=jax.ShapeDtypeStruct((B,), jnp.float32),
        mesh=plsc.VectorSubcoreMesh(core_axis_name="c", subcore_axis_name="s"),
        compiler_params=pltpu.CompilerParams(needs_layout_passes=False),
        scratch_types=[
            pltpu.VMEM((nrows * 15,), jnp.float32),
            pltpu.VMEM((19 * 32,), jnp.float32),
            pltpu.VMEM((64,), jnp.float32),
            pltpu.VMEM((nrows,), jnp.float32),
            pltpu.SemaphoreType.DMA,
        ],
    )
    out = run(xflat, consts)
    return out.reshape(B, 1)


# speedup vs baseline: 1.0130x; 1.0130x over previous
import functools

import jax
import jax.numpy as jnp
from jax import lax
from jax.experimental import pallas as pl
from jax.experimental.pallas import tpu as pltpu
from jax.experimental.pallas import tpu_sc as plsc

_NC = 2
_NS = 16
_L = 16


def _splat_i32(val):
    return jnp.full((_L,), val, dtype=jnp.int32)


_EXP2_P = (1.0000000491770782, 0.6931470026480493, 0.24022224204392606,
           0.05550712234706259, 0.009670180643285915, 0.0013260914912831122)
_MAGIC = 12582912.0
_LOG2E = 1.4426950408889634


def _sigmoid(z):
    d = 1.0 + jnp.exp(jnp.minimum(-z, 87.0))
    rc = lax.bitcast_convert_type(
        jnp.int32(0x7EF311C3) - lax.bitcast_convert_type(d, jnp.int32),
        jnp.float32)
    for _ in range(3):
        rc = rc * (2.0 - d * rc)
    return rc


def _sc_body(nrows, ngroups, x_hbm, consts_hbm, out_hbm, xv, cv, lut, ov, sem):
    wid = lax.axis_index("s") * _NC + lax.axis_index("c")
    xcp = pltpu.async_copy(
        x_hbm.at[pl.ds(wid * (nrows * 15), nrows * 15)], xv, sem)
    pltpu.sync_copy(consts_hbm, cv)

    def splat(val):
        return jnp.broadcast_to(val, (_L,))

    r18a = cv[pl.ds(18 * 32, _L)]
    r18b = cv[pl.ds(18 * 32 + _L, _L)]
    frow = [cv[pl.ds((12 + i) * 32, _L)] for i in range(6)]

    w6 = [r18a[j] for j in range(6)]
    vsp = []
    for d in range(12):
        acc = w6[0] * frow[0][d]
        for j in range(1, 6):
            acc = acc + w6[j] * frow[j][d]
        vsp.append(splat(acc))

    for half in range(2):
        ss = jnp.zeros((_L,), jnp.float32)
        aa = jnp.zeros((_L,), jnp.float32)
        for d in range(6):
            ss = ss + vsp[d] * cv[pl.ds(d * 32 + half * _L, _L)]
            aa = aa + vsp[6 + d] * cv[pl.ds((6 + d) * 32 + half * _L, _L)]
        lut[pl.ds(half * _L, _L)] = ss
        lut[pl.ds(32 + half * _L, _L)] = aa

    plsc.subcore_barrier()

    cs = r18b[25 - _L]
    for i in range(6):
        cs = cs + w6[i] * r18b[19 + i - _L]
    c16 = splat(cs)
    wd = [splat(r18a[6 + k]) if 6 + k < _L else splat(r18b[6 + k - _L])
          for k in range(13)]

    lane15 = jax.lax.iota(jnp.int32, _L) * 15
    xcp.wait()

    _UNROLL = 4

    def group(gq, carry):
        for u in range(_UNROLL):
            g = gq * _UNROLL + u
            fid = lane15 + g * (_L * 15)
            si = plsc.load_gather(xv, [fid + 13]).astype(jnp.int32)
            ai = plsc.load_gather(xv, [fid + 14]).astype(jnp.int32)
            t = [plsc.load_gather(xv, [fid + k]) * wd[k] for k in range(13)]
            t.append(plsc.load_gather(lut, [si]))
            t.append(plsc.load_gather(lut, [ai + 32]))
            t.append(c16)
            while len(t) > 1:
                t = [t[i] + t[i + 1] for i in range(0, len(t) - 1, 2)] + (
                    [t[-1]] if len(t) % 2 else [])
            z = t[0]
            ov[pl.ds(g * _L, _L)] = _sigmoid(z)
        return carry

    lax.fori_loop(0, ngroups // _UNROLL, group, 0)
    pltpu.sync_copy(ov, out_hbm.at[pl.ds(wid * nrows, nrows)])


def kernel(x, site_table, app_table, fusion_W, fusion_b, log_W, log_b):
    B = x.shape[0]
    nw = _NC * _NS
    nrows = B // nw
    ngroups = nrows // _L
    assert nrows * nw == B and ngroups * _L == nrows and ngroups % 4 == 0

    stT = jnp.zeros((6, 32), jnp.float32).at[:, :24].set(site_table.T)
    atT = app_table.T.astype(jnp.float32)
    fWp = jnp.pad(fusion_W.astype(jnp.float32), ((0, 0), (0, 20)))
    wrow = jnp.concatenate(
        [log_W[0].astype(jnp.float32), fusion_b.astype(jnp.float32),
         log_b.astype(jnp.float32), jnp.zeros((6,), jnp.float32)])
    consts = jnp.concatenate([stT, atT, fWp, wrow[None, :]], axis=0).reshape(-1)

    xflat = x.astype(jnp.float32).reshape(-1)

    run = pl.kernel(
        functools.partial(_sc_body, nrows, ngroups),
        out_type=jax.ShapeDtypeStruct((B,), jnp.float32),
        mesh=plsc.VectorSubcoreMesh(core_axis_name="c", subcore_axis_name="s"),
        compiler_params=pltpu.CompilerParams(needs_layout_passes=False),
        scratch_types=[
            pltpu.VMEM((nrows * 15,), jnp.float32),
            pltpu.VMEM((19 * 32,), jnp.float32),
            pltpu.VMEM((64,), jnp.float32),
            pltpu.VMEM((nrows,), jnp.float32),
            pltpu.SemaphoreType.DMA,
        ],
    )
    out = run(xflat, consts)
    return out.reshape(B, 1)


# speedup vs baseline: 1.0369x; 1.0236x over previous
import functools

import jax
import jax.numpy as jnp
from jax import lax
from jax.experimental import pallas as pl
from jax.experimental.pallas import tpu as pltpu
from jax.experimental.pallas import tpu_sc as plsc

_NC = 2
_NS = 16
_L = 16


def _splat_i32(val):
    return jnp.full((_L,), val, dtype=jnp.int32)


_EXP2_P = (1.0000000491770782, 0.6931470026480493, 0.24022224204392606,
           0.05550712234706259, 0.009670180643285915, 0.0013260914912831122)
_MAGIC = 12582912.0
_LOG2E = 1.4426950408889634


def _sigmoid(z):
    d = 1.0 + jnp.exp(jnp.minimum(-z, 87.0))
    rc = lax.bitcast_convert_type(
        jnp.int32(0x7EF311C3) - lax.bitcast_convert_type(d, jnp.int32),
        jnp.float32)
    for _ in range(3):
        rc = rc * (2.0 - d * rc)
    return rc


def _sc_body(nrows, ngroups, x_hbm, consts_hbm, out_hbm, xv, cv, lut, ov, spm,
             sem):
    wid = lax.axis_index("s") * _NC + lax.axis_index("c")
    xcp = pltpu.async_copy(
        x_hbm.at[pl.ds(wid * (nrows * 15), nrows * 15)], xv, sem)
    @pl.when(lax.axis_index("s") == 0)
    def _():
        pltpu.sync_copy(consts_hbm, spm)
    plsc.subcore_barrier()
    pltpu.sync_copy(spm, cv)

    def splat(val):
        return jnp.broadcast_to(val, (_L,))

    r18a = cv[pl.ds(18 * 32, _L)]
    r18b = cv[pl.ds(18 * 32 + _L, _L)]
    frow = [cv[pl.ds((12 + i) * 32, _L)] for i in range(6)]

    w6 = [r18a[j] for j in range(6)]
    vsp = []
    for d in range(12):
        acc = w6[0] * frow[0][d]
        for j in range(1, 6):
            acc = acc + w6[j] * frow[j][d]
        vsp.append(splat(acc))

    for half in range(2):
        ss = jnp.zeros((_L,), jnp.float32)
        aa = jnp.zeros((_L,), jnp.float32)
        for d in range(6):
            ss = ss + vsp[d] * cv[pl.ds(d * 32 + half * _L, _L)]
            aa = aa + vsp[6 + d] * cv[pl.ds((6 + d) * 32 + half * _L, _L)]
        lut[pl.ds(half * _L, _L)] = ss
        lut[pl.ds(32 + half * _L, _L)] = aa

    plsc.subcore_barrier()

    cs = r18b[25 - _L]
    for i in range(6):
        cs = cs + w6[i] * r18b[19 + i - _L]
    c16 = splat(cs)
    wd = [splat(r18a[6 + k]) if 6 + k < _L else splat(r18b[6 + k - _L])
          for k in range(13)]

    lane15 = jax.lax.iota(jnp.int32, _L) * 15
    xcp.wait()

    _UNROLL = 4

    def group(gq, carry):
        for u in range(_UNROLL):
            g = gq * _UNROLL + u
            fid = lane15 + g * (_L * 15)
            si = plsc.load_gather(xv, [fid + 13]).astype(jnp.int32)
            ai = plsc.load_gather(xv, [fid + 14]).astype(jnp.int32)
            t = [plsc.load_gather(xv, [fid + k]) * wd[k] for k in range(13)]
            t.append(plsc.load_gather(lut, [si]))
            t.append(plsc.load_gather(lut, [ai + 32]))
            t.append(c16)
            while len(t) > 1:
                t = [t[i] + t[i + 1] for i in range(0, len(t) - 1, 2)] + (
                    [t[-1]] if len(t) % 2 else [])
            z = t[0]
            ov[pl.ds(g * _L, _L)] = _sigmoid(z)
        return carry

    lax.fori_loop(0, ngroups // _UNROLL, group, 0)
    pltpu.sync_copy(ov, out_hbm.at[pl.ds(wid * nrows, nrows)])


def kernel(x, site_table, app_table, fusion_W, fusion_b, log_W, log_b):
    B = x.shape[0]
    nw = _NC * _NS
    nrows = B // nw
    ngroups = nrows // _L
    assert nrows * nw == B and ngroups * _L == nrows and ngroups % 4 == 0

    stT = jnp.zeros((6, 32), jnp.float32).at[:, :24].set(site_table.T)
    atT = app_table.T.astype(jnp.float32)
    fWp = jnp.pad(fusion_W.astype(jnp.float32), ((0, 0), (0, 20)))
    wrow = jnp.concatenate(
        [log_W[0].astype(jnp.float32), fusion_b.astype(jnp.float32),
         log_b.astype(jnp.float32), jnp.zeros((6,), jnp.float32)])
    consts = jnp.concatenate([stT, atT, fWp, wrow[None, :]], axis=0).reshape(-1)

    xflat = x.astype(jnp.float32).reshape(-1)

    run = pl.kernel(
        functools.partial(_sc_body, nrows, ngroups),
        out_type=jax.ShapeDtypeStruct((B,), jnp.float32),
        mesh=plsc.VectorSubcoreMesh(core_axis_name="c", subcore_axis_name="s"),
        compiler_params=pltpu.CompilerParams(needs_layout_passes=False),
        scratch_types=[
            pltpu.VMEM((nrows * 15,), jnp.float32),
            pltpu.VMEM((19 * 32,), jnp.float32),
            pltpu.VMEM((64,), jnp.float32),
            pltpu.VMEM((nrows,), jnp.float32),
            pltpu.VMEM_SHARED((19 * 32,), jnp.float32),
            pltpu.SemaphoreType.DMA,
        ],
    )
    out = run(xflat, consts)
    return out.reshape(B, 1)
